# baseline (speedup 1.0000x reference)
import functools

import jax
import jax.numpy as jnp
from jax import lax
from jax.experimental import pallas as pl
from jax.experimental.pallas import tpu as pltpu
from jax.experimental.pallas import tpu_sc as plsc

B, K, D = 16384, 1024, 256
BLK = 1024
NSPLIT = 2


def _gcn_body(lm0, ap, ac, w1p, w1c, w2p, w2c, lm_out, lmn_out):
    x = lm0[...]
    h1 = jnp.maximum(
        jnp.dot(jnp.dot(ap[...], x, preferred_element_type=jnp.float32),
                w1p[...], preferred_element_type=jnp.float32)
        + jnp.dot(jnp.dot(ac[...], x, preferred_element_type=jnp.float32),
                  w1c[...], preferred_element_type=jnp.float32),
        0.0)
    h2 = jnp.maximum(
        jnp.dot(jnp.dot(ap[...], h1, preferred_element_type=jnp.float32),
                w2p[...], preferred_element_type=jnp.float32)
        + jnp.dot(jnp.dot(ac[...], h1, preferred_element_type=jnp.float32),
                  w2c[...], preferred_element_type=jnp.float32),
        0.0)
    lm_out[...] = h2
    lmn_out[...] = jnp.sum(h2 * h2, axis=1, keepdims=True)


def _gcn(label_mat, adj_parent, adj_child, w1p, w1c, w2p, w2c):
    return pl.pallas_call(
        _gcn_body,
        out_shape=(
            jax.ShapeDtypeStruct((K, D), jnp.float32),
            jax.ShapeDtypeStruct((K, 1), jnp.float32),
        ),
    )(label_mat, adj_parent, adj_child, w1p, w1c, w2p, w2c)


def _dist_body(*refs):
    x_ref, lm_ref, lmn_ref = refs[0], refs[1], refs[2]
    ndis_ref, idx_ref = refs[-2], refs[-1]
    xb = x_ref[...]
    xn = jnp.sum(xb * xb, axis=1, keepdims=True)
    mm = lax.dot_general(xb, lm_ref[...], (((1,), (1,)), ((), ())),
                         preferred_element_type=jnp.float32)
    dist = (xn + lmn_ref[...].reshape(1, K)) - 2.0 * mm
    ndis_ref[...] = -dist
    minv = jnp.min(dist, axis=1, keepdims=True)
    kiota = lax.broadcasted_iota(jnp.int32, (BLK, K), 1)
    idx = jnp.min(jnp.where(dist == minv, kiota, K), axis=1)
    idx_ref[0, 0, ...] = idx


def _dist_chunk(x, lm, lmn, start_blk, nblk, ndis_prev):
    ins = [x, lm, lmn]
    in_specs = [
        pl.BlockSpec((BLK, D), lambda i, s=start_blk: (i + s, 0)),
        pl.BlockSpec((K, D), lambda i: (0, 0)),
        pl.BlockSpec((K, 1), lambda i: (0, 0)),
    ]
    kwargs = {}
    if ndis_prev is not None:
        ins.append(ndis_prev)
        in_specs.append(pl.BlockSpec(memory_space=pl.ANY))
        kwargs["input_output_aliases"] = {3: 0}
    return pl.pallas_call(
        _dist_body,
        grid=(nblk,),
        in_specs=in_specs,
        out_specs=(
            pl.BlockSpec((BLK, K), lambda i, s=start_blk: (i + s, 0)),
            pl.BlockSpec((1, 1, BLK), lambda i: (i, 0, 0)),
        ),
        out_shape=(
            jax.ShapeDtypeStruct((B, K), jnp.float32),
            jax.ShapeDtypeStruct((nblk, 1, BLK), jnp.int32),
        ),
        **kwargs,
    )(*ins)


_NC, _NS = 2, 16
_NW = _NC * _NS
_CH = 128


def _gather_body(nrows, table_hbm, idx_hbm, out_hbm, idx_v, rows_v,
                 gsem0, gsem1, wsem0, wsem1):
    bpw = nrows // _NW
    ch = min(_CH, bpw)
    nchunk = bpw // ch
    wid = lax.axis_index("s") * _NC + lax.axis_index("c")
    base = wid * bpw
    gsems, wsems = (gsem0, gsem1), (wsem0, wsem1)
    pltpu.sync_copy(idx_hbm.at[pl.ds(base, bpw)], idx_v)
    g = [None, None]
    w = [None, None]
    g[0] = pltpu.async_copy(table_hbm.at[idx_v.at[pl.ds(0, ch)]],
                            rows_v.at[0], gsems[0])
    for c in range(1, nchunk):
        b, pb = c % 2, (c - 1) % 2
        if w[b] is not None:
            w[b].wait()
        g[b] = pltpu.async_copy(table_hbm.at[idx_v.at[pl.ds(c * ch, ch)]],
                                rows_v.at[b], gsems[b])
        g[pb].wait()
        w[pb] = pltpu.async_copy(rows_v.at[pb],
                                 out_hbm.at[pl.ds(base + (c - 1) * ch, ch)],
                                 wsems[pb])
    lb = (nchunk - 1) % 2
    g[lb].wait()
    w[lb] = pltpu.async_copy(rows_v.at[lb],
                             out_hbm.at[pl.ds(base + (nchunk - 1) * ch, ch)],
                             wsems[lb])
    for b in range(2):
        if w[b] is not None:
            w[b].wait()


@functools.cache
def _make_sc_gather(nrows):
    bpw = nrows // _NW
    ch = min(_CH, bpw)
    return pl.kernel(
        functools.partial(_gather_body, nrows),
        out_type=jax.ShapeDtypeStruct((nrows, D), jnp.float32),
        mesh=plsc.VectorSubcoreMesh(core_axis_name="c", subcore_axis_name="s"),
        scratch_types=[
            pltpu.VMEM((bpw,), jnp.int32),
            pltpu.VMEM((2, ch, D), jnp.float32),
            pltpu.SemaphoreType.DMA,
            pltpu.SemaphoreType.DMA,
            pltpu.SemaphoreType.DMA,
            pltpu.SemaphoreType.DMA,
        ],
    )


def kernel(x, var, label_mat, adj_parent, adj_child, W1p, W1c, W2p, W2c):
    del var
    lm, lmn = _gcn(label_mat, adj_parent, adj_child, W1p, W1c, W2p, W2c)
    nblk = B // BLK
    blk_per_split = nblk // NSPLIT
    rows_per_split = B // NSPLIT
    ndis = None
    q_parts = []
    for s in range(NSPLIT):
        ndis, idx = _dist_chunk(x, lm, lmn, s * blk_per_split, blk_per_split,
                                ndis)
        q_parts.append(
            _make_sc_gather(rows_per_split)(lm, idx.reshape(rows_per_split)))
    quantized = jnp.concatenate(q_parts, axis=0) if NSPLIT > 1 else q_parts[0]
    return quantized, ndis


# speedup vs baseline: 1.1752x; 1.1752x over previous
import functools

import jax
import jax.numpy as jnp
from jax import lax
from jax.experimental import pallas as pl
from jax.experimental.pallas import tpu as pltpu
from jax.experimental.pallas import tpu_sc as plsc

B, K, D = 16384, 1024, 256
BLK = 1024
NSPLIT = 1


def _gcn_body(lm0, ap, ac, w1p, w1c, w2p, w2c, lm_out, lmn_out):
    x = lm0[...]
    h1 = jnp.maximum(
        jnp.dot(jnp.dot(ap[...], x, preferred_element_type=jnp.float32),
                w1p[...], preferred_element_type=jnp.float32)
        + jnp.dot(jnp.dot(ac[...], x, preferred_element_type=jnp.float32),
                  w1c[...], preferred_element_type=jnp.float32),
        0.0)
    h2 = jnp.maximum(
        jnp.dot(jnp.dot(ap[...], h1, preferred_element_type=jnp.float32),
                w2p[...], preferred_element_type=jnp.float32)
        + jnp.dot(jnp.dot(ac[...], h1, preferred_element_type=jnp.float32),
                  w2c[...], preferred_element_type=jnp.float32),
        0.0)
    lm_out[...] = h2
    lmn_out[...] = jnp.sum(h2 * h2, axis=1, keepdims=True)


def _gcn(label_mat, adj_parent, adj_child, w1p, w1c, w2p, w2c):
    return pl.pallas_call(
        _gcn_body,
        out_shape=(
            jax.ShapeDtypeStruct((K, D), jnp.float32),
            jax.ShapeDtypeStruct((K, 1), jnp.float32),
        ),
    )(label_mat, adj_parent, adj_child, w1p, w1c, w2p, w2c)


def _dist_body(*refs):
    x_ref, lm_ref, lmn_ref = refs[0], refs[1], refs[2]
    ndis_ref, idx_ref = refs[-2], refs[-1]
    xb = x_ref[...]
    xn = jnp.sum(xb * xb, axis=1, keepdims=True)
    mm = lax.dot_general(xb, lm_ref[...], (((1,), (1,)), ((), ())),
                         preferred_element_type=jnp.float32)
    dist = (xn + lmn_ref[...].reshape(1, K)) - 2.0 * mm
    ndis_ref[...] = -dist
    minv = jnp.min(dist, axis=1, keepdims=True)
    kiota = lax.broadcasted_iota(jnp.int32, (BLK, K), 1)
    idx = jnp.min(jnp.where(dist == minv, kiota, K), axis=1)
    idx_ref[0, 0, ...] = idx


def _dist_chunk(x, lm, lmn, start_blk, nblk, ndis_prev):
    ins = [x, lm, lmn]
    in_specs = [
        pl.BlockSpec((BLK, D), lambda i, s=start_blk: (i + s, 0)),
        pl.BlockSpec((K, D), lambda i: (0, 0)),
        pl.BlockSpec((K, 1), lambda i: (0, 0)),
    ]
    kwargs = {}
    if ndis_prev is not None:
        ins.append(ndis_prev)
        in_specs.append(pl.BlockSpec(memory_space=pl.ANY))
        kwargs["input_output_aliases"] = {3: 0}
    return pl.pallas_call(
        _dist_body,
        grid=(nblk,),
        in_specs=in_specs,
        out_specs=(
            pl.BlockSpec((BLK, K), lambda i, s=start_blk: (i + s, 0)),
            pl.BlockSpec((1, 1, BLK), lambda i: (i, 0, 0)),
        ),
        out_shape=(
            jax.ShapeDtypeStruct((B, K), jnp.float32),
            jax.ShapeDtypeStruct((nblk, 1, BLK), jnp.int32),
        ),
        **kwargs,
    )(*ins)


_NC, _NS = 2, 16
_NW = _NC * _NS
_CH = 128


def _gather_body(nrows, table_hbm, idx_hbm, out_hbm, idx_v, rows_v,
                 gsem0, gsem1, wsem0, wsem1):
    bpw = nrows // _NW
    ch = min(_CH, bpw)
    nchunk = bpw // ch
    wid = lax.axis_index("s") * _NC + lax.axis_index("c")
    base = wid * bpw
    gsems, wsems = (gsem0, gsem1), (wsem0, wsem1)
    pltpu.sync_copy(idx_hbm.at[pl.ds(base, bpw)], idx_v)
    g = [None, None]
    w = [None, None]
    g[0] = pltpu.async_copy(table_hbm.at[idx_v.at[pl.ds(0, ch)]],
                            rows_v.at[0], gsems[0])
    for c in range(1, nchunk):
        b, pb = c % 2, (c - 1) % 2
        if w[b] is not None:
            w[b].wait()
        g[b] = pltpu.async_copy(table_hbm.at[idx_v.at[pl.ds(c * ch, ch)]],
                                rows_v.at[b], gsems[b])
        g[pb].wait()
        w[pb] = pltpu.async_copy(rows_v.at[pb],
                                 out_hbm.at[pl.ds(base + (c - 1) * ch, ch)],
                                 wsems[pb])
    lb = (nchunk - 1) % 2
    g[lb].wait()
    w[lb] = pltpu.async_copy(rows_v.at[lb],
                             out_hbm.at[pl.ds(base + (nchunk - 1) * ch, ch)],
                             wsems[lb])
    for b in range(2):
        if w[b] is not None:
            w[b].wait()


@functools.cache
def _make_sc_gather(nrows):
    bpw = nrows // _NW
    ch = min(_CH, bpw)
    return pl.kernel(
        functools.partial(_gather_body, nrows),
        out_type=jax.ShapeDtypeStruct((nrows, D), jnp.float32),
        mesh=plsc.VectorSubcoreMesh(core_axis_name="c", subcore_axis_name="s"),
        scratch_types=[
            pltpu.VMEM((bpw,), jnp.int32),
            pltpu.VMEM((2, ch, D), jnp.float32),
            pltpu.SemaphoreType.DMA,
            pltpu.SemaphoreType.DMA,
            pltpu.SemaphoreType.DMA,
            pltpu.SemaphoreType.DMA,
        ],
    )


def kernel(x, var, label_mat, adj_parent, adj_child, W1p, W1c, W2p, W2c):
    del var
    lm, lmn = _gcn(label_mat, adj_parent, adj_child, W1p, W1c, W2p, W2c)
    nblk = B // BLK
    blk_per_split = nblk // NSPLIT
    rows_per_split = B // NSPLIT
    ndis = None
    q_parts = []
    for s in range(NSPLIT):
        ndis, idx = _dist_chunk(x, lm, lmn, s * blk_per_split, blk_per_split,
                                ndis)
        q_parts.append(
            _make_sc_gather(rows_per_split)(lm, idx.reshape(rows_per_split)))
    quantized = jnp.concatenate(q_parts, axis=0) if NSPLIT > 1 else q_parts[0]
    return quantized, ndis


# speedup vs baseline: 1.4485x; 1.2325x over previous
import functools

import jax
import jax.numpy as jnp
from jax import lax
from jax.experimental import pallas as pl
from jax.experimental.pallas import tpu as pltpu
from jax.experimental.pallas import tpu_sc as plsc

B, K, D = 16384, 1024, 256
BLK = 1024
NSPLIT = 1


def _gcn_body(lm0, ap, ac, w1p, w1c, w2p, w2c, lm_out, lmn_out):
    x = lm0[...]
    h1 = jnp.maximum(
        jnp.dot(jnp.dot(ap[...], x, preferred_element_type=jnp.float32),
                w1p[...], preferred_element_type=jnp.float32)
        + jnp.dot(jnp.dot(ac[...], x, preferred_element_type=jnp.float32),
                  w1c[...], preferred_element_type=jnp.float32),
        0.0)
    h2 = jnp.maximum(
        jnp.dot(jnp.dot(ap[...], h1, preferred_element_type=jnp.float32),
                w2p[...], preferred_element_type=jnp.float32)
        + jnp.dot(jnp.dot(ac[...], h1, preferred_element_type=jnp.float32),
                  w2c[...], preferred_element_type=jnp.float32),
        0.0)
    lm_out[...] = h2
    lmn_out[...] = jnp.sum(h2 * h2, axis=1, keepdims=True)


def _gcn(label_mat, adj_parent, adj_child, w1p, w1c, w2p, w2c):
    return pl.pallas_call(
        _gcn_body,
        out_shape=(
            jax.ShapeDtypeStruct((K, D), jnp.float32),
            jax.ShapeDtypeStruct((K, 1), jnp.float32),
        ),
    )(label_mat, adj_parent, adj_child, w1p, w1c, w2p, w2c)


def _dist_body(*refs):
    x_ref, lm_ref, lmn_ref = refs[0], refs[1], refs[2]
    ndis_ref, idx_ref = refs[-2], refs[-1]
    xb = x_ref[...]
    xn = jnp.sum(xb * xb, axis=1, keepdims=True)
    mm = lax.dot_general(xb, lm_ref[...], (((1,), (1,)), ((), ())),
                         preferred_element_type=jnp.float32)
    dist = (xn + lmn_ref[...].reshape(1, K)) - 2.0 * mm
    ndis_ref[...] = -dist
    minv = jnp.min(dist, axis=1, keepdims=True)
    kiota = lax.broadcasted_iota(jnp.int32, (BLK, K), 1)
    idx = jnp.min(jnp.where(dist == minv, kiota, K), axis=1)
    idx_ref[0, 0, ...] = idx


def _dist_chunk(x, lm, lmn, start_blk, nblk, ndis_prev):
    ins = [x, lm, lmn]
    in_specs = [
        pl.BlockSpec((BLK, D), lambda i, s=start_blk: (i + s, 0)),
        pl.BlockSpec((K, D), lambda i: (0, 0)),
        pl.BlockSpec((K, 1), lambda i: (0, 0)),
    ]
    kwargs = {}
    if ndis_prev is not None:
        ins.append(ndis_prev)
        in_specs.append(pl.BlockSpec(memory_space=pl.ANY))
        kwargs["input_output_aliases"] = {3: 0}
    return pl.pallas_call(
        _dist_body,
        grid=(nblk,),
        in_specs=in_specs,
        out_specs=(
            pl.BlockSpec((BLK, K), lambda i, s=start_blk: (i + s, 0)),
            pl.BlockSpec((1, 1, BLK), lambda i: (i, 0, 0)),
        ),
        out_shape=(
            jax.ShapeDtypeStruct((B, K), jnp.float32),
            jax.ShapeDtypeStruct((nblk, 1, BLK), jnp.int32),
        ),
        **kwargs,
    )(*ins)


def _dist_quant_body(*refs):
    x_ref, lm_ref, lmn_ref = refs[0], refs[1], refs[2]
    ndis_ref, q_ref = refs[-2], refs[-1]
    xb = x_ref[...]
    xn = jnp.sum(xb * xb, axis=1, keepdims=True)
    mm = lax.dot_general(xb, lm_ref[...], (((1,), (1,)), ((), ())),
                         preferred_element_type=jnp.float32)
    dist = (xn + lmn_ref[...].reshape(1, K)) - 2.0 * mm
    ndis_ref[...] = -dist
    minv = jnp.min(dist, axis=1, keepdims=True)
    kiota = lax.broadcasted_iota(jnp.int32, (BLK, K), 1)
    idx = jnp.min(jnp.where(dist == minv, kiota, K), axis=1)
    onehot = (kiota == idx[:, None]).astype(jnp.float32)
    q_ref[...] = lax.dot_general(onehot, lm_ref[...], (((1,), (0,)), ((), ())),
                                 preferred_element_type=jnp.float32)


def _dist_quant_chunk(x, lm, lmn, start_blk, nblk, ndis_prev):
    ins = [x, lm, lmn, ndis_prev]
    in_specs = [
        pl.BlockSpec((BLK, D), lambda i, s=start_blk: (i + s, 0)),
        pl.BlockSpec((K, D), lambda i: (0, 0)),
        pl.BlockSpec((K, 1), lambda i: (0, 0)),
        pl.BlockSpec(memory_space=pl.ANY),
    ]
    return pl.pallas_call(
        _dist_quant_body,
        grid=(nblk,),
        in_specs=in_specs,
        out_specs=(
            pl.BlockSpec((BLK, K), lambda i, s=start_blk: (i + s, 0)),
            pl.BlockSpec((BLK, D), lambda i: (i, 0)),
        ),
        out_shape=(
            jax.ShapeDtypeStruct((B, K), jnp.float32),
            jax.ShapeDtypeStruct((nblk * BLK, D), jnp.float32),
        ),
        input_output_aliases={3: 0},
    )(*ins)


_NC, _NS = 2, 16
_NW = _NC * _NS
_CH = 128


def _gather_body(nrows, table_hbm, idx_hbm, out_hbm, idx_v, rows_v,
                 gsem0, gsem1, wsem0, wsem1):
    bpw = nrows // _NW
    ch = min(_CH, bpw)
    nchunk = bpw // ch
    wid = lax.axis_index("s") * _NC + lax.axis_index("c")
    base = wid * bpw
    gsems, wsems = (gsem0, gsem1), (wsem0, wsem1)
    pltpu.sync_copy(idx_hbm.at[pl.ds(base, bpw)], idx_v)
    g = [None, None]
    w = [None, None]
    g[0] = pltpu.async_copy(table_hbm.at[idx_v.at[pl.ds(0, ch)]],
                            rows_v.at[0], gsems[0])
    for c in range(1, nchunk):
        b, pb = c % 2, (c - 1) % 2
        if w[b] is not None:
            w[b].wait()
        g[b] = pltpu.async_copy(table_hbm.at[idx_v.at[pl.ds(c * ch, ch)]],
                                rows_v.at[b], gsems[b])
        g[pb].wait()
        w[pb] = pltpu.async_copy(rows_v.at[pb],
                                 out_hbm.at[pl.ds(base + (c - 1) * ch, ch)],
                                 wsems[pb])
    lb = (nchunk - 1) % 2
    g[lb].wait()
    w[lb] = pltpu.async_copy(rows_v.at[lb],
                             out_hbm.at[pl.ds(base + (nchunk - 1) * ch, ch)],
                             wsems[lb])
    for b in range(2):
        if w[b] is not None:
            w[b].wait()


@functools.cache
def _make_sc_gather(nrows):
    bpw = nrows // _NW
    ch = min(_CH, bpw)
    return pl.kernel(
        functools.partial(_gather_body, nrows),
        out_type=jax.ShapeDtypeStruct((nrows, D), jnp.float32),
        mesh=plsc.VectorSubcoreMesh(core_axis_name="c", subcore_axis_name="s"),
        scratch_types=[
            pltpu.VMEM((bpw,), jnp.int32),
            pltpu.VMEM((2, ch, D), jnp.float32),
            pltpu.SemaphoreType.DMA,
            pltpu.SemaphoreType.DMA,
            pltpu.SemaphoreType.DMA,
            pltpu.SemaphoreType.DMA,
        ],
    )


NA = 8


def kernel(x, var, label_mat, adj_parent, adj_child, W1p, W1c, W2p, W2c):
    del var
    lm, lmn = _gcn(label_mat, adj_parent, adj_child, W1p, W1c, W2p, W2c)
    nblk = B // BLK
    rows_a = NA * BLK
    ndis, idx_a = _dist_chunk(x, lm, lmn, 0, NA, None)
    q_a = _make_sc_gather(rows_a)(lm, idx_a.reshape(rows_a))
    ndis, q_b = _dist_quant_chunk(x, lm, lmn, NA, nblk - NA, ndis)
    return jnp.concatenate([q_a, q_b], axis=0), ndis


# speedup vs baseline: 1.7628x; 1.2170x over previous
import functools

import jax
import jax.numpy as jnp
from jax import lax
from jax.experimental import pallas as pl
from jax.experimental.pallas import tpu as pltpu
from jax.experimental.pallas import tpu_sc as plsc

B, K, D = 16384, 1024, 256
BLK = 1024
NSPLIT = 1


def _gcn_body(lm0, ap, ac, w1p, w1c, w2p, w2c, lm_out, lmn_out):
    x = lm0[...]
    h1 = jnp.maximum(
        jnp.dot(jnp.dot(ap[...], x, preferred_element_type=jnp.float32),
                w1p[...], preferred_element_type=jnp.float32)
        + jnp.dot(jnp.dot(ac[...], x, preferred_element_type=jnp.float32),
                  w1c[...], preferred_element_type=jnp.float32),
        0.0)
    h2 = jnp.maximum(
        jnp.dot(jnp.dot(ap[...], h1, preferred_element_type=jnp.float32),
                w2p[...], preferred_element_type=jnp.float32)
        + jnp.dot(jnp.dot(ac[...], h1, preferred_element_type=jnp.float32),
                  w2c[...], preferred_element_type=jnp.float32),
        0.0)
    lm_out[...] = h2
    lmn_out[...] = jnp.sum(h2 * h2, axis=1, keepdims=True)


def _gcn(label_mat, adj_parent, adj_child, w1p, w1c, w2p, w2c):
    return pl.pallas_call(
        _gcn_body,
        out_shape=(
            jax.ShapeDtypeStruct((K, D), jnp.float32),
            jax.ShapeDtypeStruct((K, 1), jnp.float32),
        ),
    )(label_mat, adj_parent, adj_child, w1p, w1c, w2p, w2c)


def _dist_body(*refs):
    x_ref, lm_ref, lmn_ref = refs[0], refs[1], refs[2]
    ndis_ref, idx_ref = refs[-2], refs[-1]
    xb = x_ref[...]
    xn = jnp.sum(xb * xb, axis=1, keepdims=True)
    mm = lax.dot_general(xb, lm_ref[...], (((1,), (1,)), ((), ())),
                         preferred_element_type=jnp.float32)
    dist = (xn + lmn_ref[...].reshape(1, K)) - 2.0 * mm
    ndis_ref[...] = -dist
    minv = jnp.min(dist, axis=1, keepdims=True)
    kiota = lax.broadcasted_iota(jnp.int32, (BLK, K), 1)
    idx = jnp.min(jnp.where(dist == minv, kiota, K), axis=1)
    idx_ref[0, 0, ...] = idx


def _dist_chunk(x, lm, lmn, start_blk, nblk, ndis_prev):
    ins = [x, lm, lmn]
    in_specs = [
        pl.BlockSpec((BLK, D), lambda i, s=start_blk: (i + s, 0)),
        pl.BlockSpec((K, D), lambda i: (0, 0)),
        pl.BlockSpec((K, 1), lambda i: (0, 0)),
    ]
    kwargs = {}
    if ndis_prev is not None:
        ins.append(ndis_prev)
        in_specs.append(pl.BlockSpec(memory_space=pl.ANY))
        kwargs["input_output_aliases"] = {3: 0}
    return pl.pallas_call(
        _dist_body,
        grid=(nblk,),
        in_specs=in_specs,
        out_specs=(
            pl.BlockSpec((BLK, K), lambda i, s=start_blk: (i + s, 0)),
            pl.BlockSpec((1, 1, BLK), lambda i: (i, 0, 0)),
        ),
        out_shape=(
            jax.ShapeDtypeStruct((B, K), jnp.float32),
            jax.ShapeDtypeStruct((nblk, 1, BLK), jnp.int32),
        ),
        **kwargs,
    )(*ins)


def _dist_quant_body(*refs):
    x_ref, lm_ref, lmn_ref = refs[0], refs[1], refs[2]
    ndis_ref, q_ref = refs[-2], refs[-1]
    xb = x_ref[...]
    xn = jnp.sum(xb * xb, axis=1, keepdims=True)
    mm = lax.dot_general(xb, lm_ref[...], (((1,), (1,)), ((), ())),
                         preferred_element_type=jnp.float32)
    dist = (xn + lmn_ref[...].reshape(1, K)) - 2.0 * mm
    ndis_ref[...] = -dist
    minv = jnp.min(dist, axis=1, keepdims=True)
    kiota = lax.broadcasted_iota(jnp.int32, (BLK, K), 1)
    idx = jnp.min(jnp.where(dist == minv, kiota, K), axis=1)
    onehot = (kiota == idx[:, None]).astype(jnp.float32)
    q_ref[...] = lax.dot_general(onehot, lm_ref[...], (((1,), (0,)), ((), ())),
                                 preferred_element_type=jnp.float32)


def _dist_quant_chunk(x, lm, lmn, start_blk, nblk, ndis_prev):
    ins = [x, lm, lmn, ndis_prev]
    in_specs = [
        pl.BlockSpec((BLK, D), lambda i, s=start_blk: (i + s, 0)),
        pl.BlockSpec((K, D), lambda i: (0, 0)),
        pl.BlockSpec((K, 1), lambda i: (0, 0)),
        pl.BlockSpec(memory_space=pl.ANY),
    ]
    return pl.pallas_call(
        _dist_quant_body,
        grid=(nblk,),
        in_specs=in_specs,
        out_specs=(
            pl.BlockSpec((BLK, K), lambda i, s=start_blk: (i + s, 0)),
            pl.BlockSpec((BLK, D), lambda i: (i, 0)),
        ),
        out_shape=(
            jax.ShapeDtypeStruct((B, K), jnp.float32),
            jax.ShapeDtypeStruct((nblk * BLK, D), jnp.float32),
        ),
        input_output_aliases={3: 0},
    )(*ins)


_NC, _NS = 2, 16
_NW = _NC * _NS
_CH = 128


def _gather_body(nrows, table_hbm, idx_hbm, out_hbm, idx_v, rows_v,
                 gsem0, gsem1, wsem0, wsem1):
    bpw = nrows // _NW
    ch = min(_CH, bpw)
    nchunk = bpw // ch
    wid = lax.axis_index("s") * _NC + lax.axis_index("c")
    base = wid * bpw
    gsems, wsems = (gsem0, gsem1), (wsem0, wsem1)
    pltpu.sync_copy(idx_hbm.at[pl.ds(base, bpw)], idx_v)
    g = [None, None]
    w = [None, None]
    g[0] = pltpu.async_copy(table_hbm.at[idx_v.at[pl.ds(0, ch)]],
                            rows_v.at[0], gsems[0])
    for c in range(1, nchunk):
        b, pb = c % 2, (c - 1) % 2
        if w[b] is not None:
            w[b].wait()
        g[b] = pltpu.async_copy(table_hbm.at[idx_v.at[pl.ds(c * ch, ch)]],
                                rows_v.at[b], gsems[b])
        g[pb].wait()
        w[pb] = pltpu.async_copy(rows_v.at[pb],
                                 out_hbm.at[pl.ds(base + (c - 1) * ch, ch)],
                                 wsems[pb])
    lb = (nchunk - 1) % 2
    g[lb].wait()
    w[lb] = pltpu.async_copy(rows_v.at[lb],
                             out_hbm.at[pl.ds(base + (nchunk - 1) * ch, ch)],
                             wsems[lb])
    for b in range(2):
        if w[b] is not None:
            w[b].wait()


@functools.cache
def _make_sc_gather(nrows):
    bpw = nrows // _NW
    ch = min(_CH, bpw)
    return pl.kernel(
        functools.partial(_gather_body, nrows),
        out_type=jax.ShapeDtypeStruct((nrows, D), jnp.float32),
        mesh=plsc.VectorSubcoreMesh(core_axis_name="c", subcore_axis_name="s"),
        scratch_types=[
            pltpu.VMEM((bpw,), jnp.int32),
            pltpu.VMEM((2, ch, D), jnp.float32),
            pltpu.SemaphoreType.DMA,
            pltpu.SemaphoreType.DMA,
            pltpu.SemaphoreType.DMA,
            pltpu.SemaphoreType.DMA,
        ],
    )


NA = 4


def kernel(x, var, label_mat, adj_parent, adj_child, W1p, W1c, W2p, W2c):
    del var
    lm, lmn = _gcn(label_mat, adj_parent, adj_child, W1p, W1c, W2p, W2c)
    nblk = B // BLK
    rows_a = NA * BLK
    ndis, idx_a = _dist_chunk(x, lm, lmn, 0, NA, None)
    q_a = _make_sc_gather(rows_a)(lm, idx_a.reshape(rows_a))
    ndis, q_b = _dist_quant_chunk(x, lm, lmn, NA, nblk - NA, ndis)
    return jnp.concatenate([q_a, q_b], axis=0), ndis


# speedup vs baseline: 1.8680x; 1.0597x over previous
import functools

import jax
import jax.numpy as jnp
from jax import lax
from jax.experimental import pallas as pl
from jax.experimental.pallas import tpu as pltpu
from jax.experimental.pallas import tpu_sc as plsc

B, K, D = 16384, 1024, 256
BLK = 1024
NSPLIT = 1


def _gcn_body(lm0, ap, ac, w1p, w1c, w2p, w2c, lm_out, lmn_out):
    x = lm0[...]
    h1 = jnp.maximum(
        jnp.dot(jnp.dot(ap[...], x, preferred_element_type=jnp.float32),
                w1p[...], preferred_element_type=jnp.float32)
        + jnp.dot(jnp.dot(ac[...], x, preferred_element_type=jnp.float32),
                  w1c[...], preferred_element_type=jnp.float32),
        0.0)
    h2 = jnp.maximum(
        jnp.dot(jnp.dot(ap[...], h1, preferred_element_type=jnp.float32),
                w2p[...], preferred_element_type=jnp.float32)
        + jnp.dot(jnp.dot(ac[...], h1, preferred_element_type=jnp.float32),
                  w2c[...], preferred_element_type=jnp.float32),
        0.0)
    lm_out[...] = h2
    lmn_out[...] = jnp.sum(h2 * h2, axis=1, keepdims=True)


def _gcn(label_mat, adj_parent, adj_child, w1p, w1c, w2p, w2c):
    return pl.pallas_call(
        _gcn_body,
        out_shape=(
            jax.ShapeDtypeStruct((K, D), jnp.float32),
            jax.ShapeDtypeStruct((K, 1), jnp.float32),
        ),
    )(label_mat, adj_parent, adj_child, w1p, w1c, w2p, w2c)


def _dist_body(*refs):
    x_ref, lm_ref, lmn_ref = refs[0], refs[1], refs[2]
    ndis_ref, idx_ref = refs[-2], refs[-1]
    xb = x_ref[...]
    xn = jnp.sum(xb * xb, axis=1, keepdims=True)
    mm = lax.dot_general(xb, lm_ref[...], (((1,), (1,)), ((), ())),
                         preferred_element_type=jnp.float32)
    dist = (xn + lmn_ref[...].reshape(1, K)) - 2.0 * mm
    ndis_ref[...] = -dist
    minv = jnp.min(dist, axis=1, keepdims=True)
    kiota = lax.broadcasted_iota(jnp.int32, (BLK, K), 1)
    idx = jnp.min(jnp.where(dist == minv, kiota, K), axis=1)
    idx_ref[0, 0, ...] = idx


def _dist_chunk(x, lm, lmn, start_blk, nblk, ndis_prev):
    ins = [x, lm, lmn]
    in_specs = [
        pl.BlockSpec((BLK, D), lambda i, s=start_blk: (i + s, 0)),
        pl.BlockSpec((K, D), lambda i: (0, 0)),
        pl.BlockSpec((K, 1), lambda i: (0, 0)),
    ]
    kwargs = {}
    if ndis_prev is not None:
        ins.append(ndis_prev)
        in_specs.append(pl.BlockSpec(memory_space=pl.ANY))
        kwargs["input_output_aliases"] = {3: 0}
    return pl.pallas_call(
        _dist_body,
        grid=(nblk,),
        in_specs=in_specs,
        out_specs=(
            pl.BlockSpec((BLK, K), lambda i, s=start_blk: (i + s, 0)),
            pl.BlockSpec((1, 1, BLK), lambda i: (i, 0, 0)),
        ),
        out_shape=(
            jax.ShapeDtypeStruct((B, K), jnp.float32),
            jax.ShapeDtypeStruct((nblk, 1, BLK), jnp.int32),
        ),
        **kwargs,
    )(*ins)


def _dist_quant_body(*refs):
    x_ref, lm_ref, lmn_ref = refs[0], refs[1], refs[2]
    ndis_ref, q_ref = refs[-2], refs[-1]
    xb = x_ref[...]
    xn = jnp.sum(xb * xb, axis=1, keepdims=True)
    mm = lax.dot_general(xb, lm_ref[...], (((1,), (1,)), ((), ())),
                         preferred_element_type=jnp.float32)
    dist = (xn + lmn_ref[...].reshape(1, K)) - 2.0 * mm
    ndis_ref[...] = -dist
    minv = jnp.min(dist, axis=1, keepdims=True)
    kiota = lax.broadcasted_iota(jnp.int32, (BLK, K), 1)
    idx = jnp.min(jnp.where(dist == minv, kiota, K), axis=1)
    onehot = (kiota == idx[:, None]).astype(jnp.float32)
    q_ref[...] = lax.dot_general(onehot, lm_ref[...], (((1,), (0,)), ((), ())),
                                 preferred_element_type=jnp.float32)


def _dist_quant_chunk(x, lm, lmn, start_blk, nblk, ndis_prev):
    ins = [x, lm, lmn, ndis_prev]
    in_specs = [
        pl.BlockSpec((BLK, D), lambda i, s=start_blk: (i + s, 0)),
        pl.BlockSpec((K, D), lambda i: (0, 0)),
        pl.BlockSpec((K, 1), lambda i: (0, 0)),
        pl.BlockSpec(memory_space=pl.ANY),
    ]
    return pl.pallas_call(
        _dist_quant_body,
        grid=(nblk,),
        in_specs=in_specs,
        out_specs=(
            pl.BlockSpec((BLK, K), lambda i, s=start_blk: (i + s, 0)),
            pl.BlockSpec((BLK, D), lambda i: (i, 0)),
        ),
        out_shape=(
            jax.ShapeDtypeStruct((B, K), jnp.float32),
            jax.ShapeDtypeStruct((nblk * BLK, D), jnp.float32),
        ),
        input_output_aliases={3: 0},
    )(*ins)


_NC, _NS = 2, 16
_NW = _NC * _NS
_CH = 128


def _gather_body(nrows, table_hbm, idx_hbm, out_hbm, idx_v, rows_v,
                 gsem0, gsem1, wsem0, wsem1):
    bpw = nrows // _NW
    ch = min(_CH, bpw)
    nchunk = bpw // ch
    wid = lax.axis_index("s") * _NC + lax.axis_index("c")
    base = wid * bpw
    gsems, wsems = (gsem0, gsem1), (wsem0, wsem1)
    pltpu.sync_copy(idx_hbm.at[pl.ds(base, bpw)], idx_v)
    g = [None, None]
    w = [None, None]
    g[0] = pltpu.async_copy(table_hbm.at[idx_v.at[pl.ds(0, ch)]],
                            rows_v.at[0], gsems[0])
    for c in range(1, nchunk):
        b, pb = c % 2, (c - 1) % 2
        if w[b] is not None:
            w[b].wait()
        g[b] = pltpu.async_copy(table_hbm.at[idx_v.at[pl.ds(c * ch, ch)]],
                                rows_v.at[b], gsems[b])
        g[pb].wait()
        w[pb] = pltpu.async_copy(rows_v.at[pb],
                                 out_hbm.at[pl.ds(base + (c - 1) * ch, ch)],
                                 wsems[pb])
    lb = (nchunk - 1) % 2
    g[lb].wait()
    w[lb] = pltpu.async_copy(rows_v.at[lb],
                             out_hbm.at[pl.ds(base + (nchunk - 1) * ch, ch)],
                             wsems[lb])
    for b in range(2):
        if w[b] is not None:
            w[b].wait()


@functools.cache
def _make_sc_gather(nrows):
    bpw = nrows // _NW
    ch = min(_CH, bpw)
    return pl.kernel(
        functools.partial(_gather_body, nrows),
        out_type=jax.ShapeDtypeStruct((nrows, D), jnp.float32),
        mesh=plsc.VectorSubcoreMesh(core_axis_name="c", subcore_axis_name="s"),
        scratch_types=[
            pltpu.VMEM((bpw,), jnp.int32),
            pltpu.VMEM((2, ch, D), jnp.float32),
            pltpu.SemaphoreType.DMA,
            pltpu.SemaphoreType.DMA,
            pltpu.SemaphoreType.DMA,
            pltpu.SemaphoreType.DMA,
        ],
    )


NA = 2


def kernel(x, var, label_mat, adj_parent, adj_child, W1p, W1c, W2p, W2c):
    del var
    lm, lmn = _gcn(label_mat, adj_parent, adj_child, W1p, W1c, W2p, W2c)
    nblk = B // BLK
    rows_a = NA * BLK
    ndis, idx_a = _dist_chunk(x, lm, lmn, 0, NA, None)
    q_a = _make_sc_gather(rows_a)(lm, idx_a.reshape(rows_a))
    ndis, q_b = _dist_quant_chunk(x, lm, lmn, NA, nblk - NA, ndis)
    return jnp.concatenate([q_a, q_b], axis=0), ndis


# speedup vs baseline: 1.9006x; 1.0174x over previous
import functools

import jax
import jax.numpy as jnp
from jax import lax
from jax.experimental import pallas as pl
from jax.experimental.pallas import tpu as pltpu
from jax.experimental.pallas import tpu_sc as plsc

B, K, D = 16384, 1024, 256
BLK = 1024
NSPLIT = 1


def _gcn_body(lm0, ap, ac, w1p, w1c, w2p, w2c, lm_out, lmn_out):
    x = lm0[...]
    h1 = jnp.maximum(
        jnp.dot(jnp.dot(ap[...], x, preferred_element_type=jnp.float32),
                w1p[...], preferred_element_type=jnp.float32)
        + jnp.dot(jnp.dot(ac[...], x, preferred_element_type=jnp.float32),
                  w1c[...], preferred_element_type=jnp.float32),
        0.0)
    h2 = jnp.maximum(
        jnp.dot(jnp.dot(ap[...], h1, preferred_element_type=jnp.float32),
                w2p[...], preferred_element_type=jnp.float32)
        + jnp.dot(jnp.dot(ac[...], h1, preferred_element_type=jnp.float32),
                  w2c[...], preferred_element_type=jnp.float32),
        0.0)
    lm_out[...] = h2
    lmn_out[...] = jnp.sum(h2 * h2, axis=1, keepdims=True)


def _gcn(label_mat, adj_parent, adj_child, w1p, w1c, w2p, w2c):
    return pl.pallas_call(
        _gcn_body,
        out_shape=(
            jax.ShapeDtypeStruct((K, D), jnp.float32),
            jax.ShapeDtypeStruct((K, 1), jnp.float32),
        ),
    )(label_mat, adj_parent, adj_child, w1p, w1c, w2p, w2c)


def _dist_body(*refs):
    x_ref, lm_ref, lmn_ref = refs[0], refs[1], refs[2]
    ndis_ref, idx_ref = refs[-2], refs[-1]
    xb = x_ref[...]
    xn = jnp.sum(xb * xb, axis=1, keepdims=True)
    mm = lax.dot_general(xb, lm_ref[...], (((1,), (1,)), ((), ())),
                         preferred_element_type=jnp.float32)
    dist = (xn + lmn_ref[...].reshape(1, K)) - 2.0 * mm
    ndis_ref[...] = -dist
    minv = jnp.min(dist, axis=1, keepdims=True)
    kiota = lax.broadcasted_iota(jnp.int32, (BLK, K), 1)
    idx = jnp.min(jnp.where(dist == minv, kiota, K), axis=1)
    idx_ref[0, 0, ...] = idx


def _dist_chunk(x, lm, lmn, start_blk, nblk, ndis_prev):
    ins = [x, lm, lmn]
    in_specs = [
        pl.BlockSpec((BLK, D), lambda i, s=start_blk: (i + s, 0)),
        pl.BlockSpec((K, D), lambda i: (0, 0)),
        pl.BlockSpec((K, 1), lambda i: (0, 0)),
    ]
    kwargs = {}
    if ndis_prev is not None:
        ins.append(ndis_prev)
        in_specs.append(pl.BlockSpec(memory_space=pl.ANY))
        kwargs["input_output_aliases"] = {3: 0}
    return pl.pallas_call(
        _dist_body,
        grid=(nblk,),
        in_specs=in_specs,
        out_specs=(
            pl.BlockSpec((BLK, K), lambda i, s=start_blk: (i + s, 0)),
            pl.BlockSpec((1, 1, BLK), lambda i: (i, 0, 0)),
        ),
        out_shape=(
            jax.ShapeDtypeStruct((B, K), jnp.float32),
            jax.ShapeDtypeStruct((nblk, 1, BLK), jnp.int32),
        ),
        **kwargs,
    )(*ins)


def _dist_quant_body(*refs):
    x_ref, lm_ref, lmn_ref = refs[0], refs[1], refs[2]
    ndis_ref, q_ref = refs[-2], refs[-1]
    xb = x_ref[...]
    xn = jnp.sum(xb * xb, axis=1, keepdims=True)
    mm = lax.dot_general(xb, lm_ref[...], (((1,), (1,)), ((), ())),
                         preferred_element_type=jnp.float32)
    dist = (xn + lmn_ref[...].reshape(1, K)) - 2.0 * mm
    ndis_ref[...] = -dist
    minv = jnp.min(dist, axis=1, keepdims=True)
    kiota = lax.broadcasted_iota(jnp.int32, (BLK, K), 1)
    idx = jnp.min(jnp.where(dist == minv, kiota, K), axis=1)
    onehot = (kiota == idx[:, None]).astype(jnp.float32)
    q_ref[...] = lax.dot_general(onehot, lm_ref[...], (((1,), (0,)), ((), ())),
                                 preferred_element_type=jnp.float32)


def _dist_quant_chunk(x, lm, lmn, start_blk, nblk, ndis_prev):
    ins = [x, lm, lmn, ndis_prev]
    in_specs = [
        pl.BlockSpec((BLK, D), lambda i, s=start_blk: (i + s, 0)),
        pl.BlockSpec((K, D), lambda i: (0, 0)),
        pl.BlockSpec((K, 1), lambda i: (0, 0)),
        pl.BlockSpec(memory_space=pl.ANY),
    ]
    return pl.pallas_call(
        _dist_quant_body,
        grid=(nblk,),
        in_specs=in_specs,
        out_specs=(
            pl.BlockSpec((BLK, K), lambda i, s=start_blk: (i + s, 0)),
            pl.BlockSpec((BLK, D), lambda i: (i, 0)),
        ),
        out_shape=(
            jax.ShapeDtypeStruct((B, K), jnp.float32),
            jax.ShapeDtypeStruct((nblk * BLK, D), jnp.float32),
        ),
        input_output_aliases={3: 0},
    )(*ins)


_NC, _NS = 2, 16
_NW = _NC * _NS
_CH = 128


def _gather_body(nrows, table_hbm, idx_hbm, out_hbm, idx_v, rows_v,
                 gsem0, gsem1, wsem0, wsem1):
    bpw = nrows // _NW
    ch = min(_CH, bpw)
    nchunk = bpw // ch
    wid = lax.axis_index("s") * _NC + lax.axis_index("c")
    base = wid * bpw
    gsems, wsems = (gsem0, gsem1), (wsem0, wsem1)
    pltpu.sync_copy(idx_hbm.at[pl.ds(base, bpw)], idx_v)
    g = [None, None]
    w = [None, None]
    g[0] = pltpu.async_copy(table_hbm.at[idx_v.at[pl.ds(0, ch)]],
                            rows_v.at[0], gsems[0])
    for c in range(1, nchunk):
        b, pb = c % 2, (c - 1) % 2
        if w[b] is not None:
            w[b].wait()
        g[b] = pltpu.async_copy(table_hbm.at[idx_v.at[pl.ds(c * ch, ch)]],
                                rows_v.at[b], gsems[b])
        g[pb].wait()
        w[pb] = pltpu.async_copy(rows_v.at[pb],
                                 out_hbm.at[pl.ds(base + (c - 1) * ch, ch)],
                                 wsems[pb])
    lb = (nchunk - 1) % 2
    g[lb].wait()
    w[lb] = pltpu.async_copy(rows_v.at[lb],
                             out_hbm.at[pl.ds(base + (nchunk - 1) * ch, ch)],
                             wsems[lb])
    for b in range(2):
        if w[b] is not None:
            w[b].wait()


@functools.cache
def _make_sc_gather(nrows):
    bpw = nrows // _NW
    ch = min(_CH, bpw)
    return pl.kernel(
        functools.partial(_gather_body, nrows),
        out_type=jax.ShapeDtypeStruct((nrows, D), jnp.float32),
        mesh=plsc.VectorSubcoreMesh(core_axis_name="c", subcore_axis_name="s"),
        scratch_types=[
            pltpu.VMEM((bpw,), jnp.int32),
            pltpu.VMEM((2, ch, D), jnp.float32),
            pltpu.SemaphoreType.DMA,
            pltpu.SemaphoreType.DMA,
            pltpu.SemaphoreType.DMA,
            pltpu.SemaphoreType.DMA,
        ],
    )


NA = 1


def kernel(x, var, label_mat, adj_parent, adj_child, W1p, W1c, W2p, W2c):
    del var
    lm, lmn = _gcn(label_mat, adj_parent, adj_child, W1p, W1c, W2p, W2c)
    nblk = B // BLK
    rows_a = NA * BLK
    ndis, idx_a = _dist_chunk(x, lm, lmn, 0, NA, None)
    q_a = _make_sc_gather(rows_a)(lm, idx_a.reshape(rows_a))
    ndis, q_b = _dist_quant_chunk(x, lm, lmn, NA, nblk - NA, ndis)
    return jnp.concatenate([q_a, q_b], axis=0), ndis


# speedup vs baseline: 1.9038x; 1.0017x over previous
import functools

import jax
import jax.numpy as jnp
from jax import lax
from jax.experimental import pallas as pl
from jax.experimental.pallas import tpu as pltpu
from jax.experimental.pallas import tpu_sc as plsc

B, K, D = 16384, 1024, 256
BLK = 1024
NSPLIT = 1


def _gcn_body(lm0, ap, ac, w1p, w1c, w2p, w2c, lm_out, lm2_out, lmn_out):
    x = lm0[...]
    h1 = jnp.maximum(
        jnp.dot(jnp.dot(ap[...], x, preferred_element_type=jnp.float32),
                w1p[...], preferred_element_type=jnp.float32)
        + jnp.dot(jnp.dot(ac[...], x, preferred_element_type=jnp.float32),
                  w1c[...], preferred_element_type=jnp.float32),
        0.0)
    h2 = jnp.maximum(
        jnp.dot(jnp.dot(ap[...], h1, preferred_element_type=jnp.float32),
                w2p[...], preferred_element_type=jnp.float32)
        + jnp.dot(jnp.dot(ac[...], h1, preferred_element_type=jnp.float32),
                  w2c[...], preferred_element_type=jnp.float32),
        0.0)
    lm_out[...] = h2
    lm2_out[...] = -2.0 * h2
    lmn_out[...] = jnp.sum(h2 * h2, axis=1, keepdims=True)


def _gcn(label_mat, adj_parent, adj_child, w1p, w1c, w2p, w2c):
    return pl.pallas_call(
        _gcn_body,
        out_shape=(
            jax.ShapeDtypeStruct((K, D), jnp.float32),
            jax.ShapeDtypeStruct((K, D), jnp.float32),
            jax.ShapeDtypeStruct((K, 1), jnp.float32),
        ),
    )(label_mat, adj_parent, adj_child, w1p, w1c, w2p, w2c)


def _ndis_argmax(xb, lm2_ref, lmn_ref):
    xn = jnp.sum(xb * xb, axis=1, keepdims=True)
    mm2 = lax.dot_general(xb, lm2_ref[...], (((1,), (1,)), ((), ())),
                          preferred_element_type=jnp.float32)
    ndis = (mm2 - xn) - lmn_ref[...].reshape(1, K)
    maxv = jnp.max(ndis, axis=1, keepdims=True)
    kiota = lax.broadcasted_iota(jnp.int32, (BLK, K), 1)
    idx = jnp.min(jnp.where(ndis == maxv, kiota, K), axis=1)
    return ndis, idx, kiota


def _dist_body(*refs):
    x_ref, lm2_ref, lmn_ref = refs[0], refs[1], refs[2]
    ndis_ref, idx_ref = refs[-2], refs[-1]
    ndis, idx, _ = _ndis_argmax(x_ref[...], lm2_ref, lmn_ref)
    ndis_ref[...] = ndis
    idx_ref[0, 0, ...] = idx


def _dist_chunk(x, lm2, lmn, start_blk, nblk, ndis_prev):
    ins = [x, lm2, lmn]
    in_specs = [
        pl.BlockSpec((BLK, D), lambda i, s=start_blk: (i + s, 0)),
        pl.BlockSpec((K, D), lambda i: (0, 0)),
        pl.BlockSpec((K, 1), lambda i: (0, 0)),
    ]
    kwargs = {}
    if ndis_prev is not None:
        ins.append(ndis_prev)
        in_specs.append(pl.BlockSpec(memory_space=pl.ANY))
        kwargs["input_output_aliases"] = {3: 0}
    return pl.pallas_call(
        _dist_body,
        grid=(nblk,),
        in_specs=in_specs,
        out_specs=(
            pl.BlockSpec((BLK, K), lambda i, s=start_blk: (i + s, 0)),
            pl.BlockSpec((1, 1, BLK), lambda i: (i, 0, 0)),
        ),
        out_shape=(
            jax.ShapeDtypeStruct((B, K), jnp.float32),
            jax.ShapeDtypeStruct((nblk, 1, BLK), jnp.int32),
        ),
        **kwargs,
    )(*ins)


def _dist_quant_body(*refs):
    x_ref, lm2_ref, lmn_ref, lm_ref = refs[0], refs[1], refs[2], refs[3]
    ndis_ref, q_ref = refs[-2], refs[-1]
    ndis, idx, kiota = _ndis_argmax(x_ref[...], lm2_ref, lmn_ref)
    ndis_ref[...] = ndis
    onehot = (kiota == idx[:, None]).astype(jnp.float32)
    q_ref[...] = lax.dot_general(onehot, lm_ref[...], (((1,), (0,)), ((), ())),
                                 preferred_element_type=jnp.float32)


def _dist_quant_chunk(x, lm2, lmn, lm, start_blk, nblk, ndis_prev):
    ins = [x, lm2, lmn, lm, ndis_prev]
    in_specs = [
        pl.BlockSpec((BLK, D), lambda i, s=start_blk: (i + s, 0)),
        pl.BlockSpec((K, D), lambda i: (0, 0)),
        pl.BlockSpec((K, 1), lambda i: (0, 0)),
        pl.BlockSpec((K, D), lambda i: (0, 0)),
        pl.BlockSpec(memory_space=pl.ANY),
    ]
    return pl.pallas_call(
        _dist_quant_body,
        grid=(nblk,),
        in_specs=in_specs,
        out_specs=(
            pl.BlockSpec((BLK, K), lambda i, s=start_blk: (i + s, 0)),
            pl.BlockSpec((BLK, D), lambda i: (i, 0)),
        ),
        out_shape=(
            jax.ShapeDtypeStruct((B, K), jnp.float32),
            jax.ShapeDtypeStruct((nblk * BLK, D), jnp.float32),
        ),
        input_output_aliases={4: 0},
    )(*ins)


_NC, _NS = 2, 16
_NW = _NC * _NS
_CH = 128


def _gather_body(nrows, table_hbm, idx_hbm, out_hbm, idx_v, rows_v,
                 gsem0, gsem1, wsem0, wsem1):
    bpw = nrows // _NW
    ch = min(_CH, bpw)
    nchunk = bpw // ch
    wid = lax.axis_index("s") * _NC + lax.axis_index("c")
    base = wid * bpw
    gsems, wsems = (gsem0, gsem1), (wsem0, wsem1)
    pltpu.sync_copy(idx_hbm.at[pl.ds(base, bpw)], idx_v)
    g = [None, None]
    w = [None, None]
    g[0] = pltpu.async_copy(table_hbm.at[idx_v.at[pl.ds(0, ch)]],
                            rows_v.at[0], gsems[0])
    for c in range(1, nchunk):
        b, pb = c % 2, (c - 1) % 2
        if w[b] is not None:
            w[b].wait()
        g[b] = pltpu.async_copy(table_hbm.at[idx_v.at[pl.ds(c * ch, ch)]],
                                rows_v.at[b], gsems[b])
        g[pb].wait()
        w[pb] = pltpu.async_copy(rows_v.at[pb],
                                 out_hbm.at[pl.ds(base + (c - 1) * ch, ch)],
                                 wsems[pb])
    lb = (nchunk - 1) % 2
    g[lb].wait()
    w[lb] = pltpu.async_copy(rows_v.at[lb],
                             out_hbm.at[pl.ds(base + (nchunk - 1) * ch, ch)],
                             wsems[lb])
    for b in range(2):
        if w[b] is not None:
            w[b].wait()


@functools.cache
def _make_sc_gather(nrows):
    bpw = nrows // _NW
    ch = min(_CH, bpw)
    return pl.kernel(
        functools.partial(_gather_body, nrows),
        out_type=jax.ShapeDtypeStruct((nrows, D), jnp.float32),
        mesh=plsc.VectorSubcoreMesh(core_axis_name="c", subcore_axis_name="s"),
        scratch_types=[
            pltpu.VMEM((bpw,), jnp.int32),
            pltpu.VMEM((2, ch, D), jnp.float32),
            pltpu.SemaphoreType.DMA,
            pltpu.SemaphoreType.DMA,
            pltpu.SemaphoreType.DMA,
            pltpu.SemaphoreType.DMA,
        ],
    )


NA = 1


def kernel(x, var, label_mat, adj_parent, adj_child, W1p, W1c, W2p, W2c):
    del var
    lm, lm2, lmn = _gcn(label_mat, adj_parent, adj_child, W1p, W1c, W2p, W2c)
    nblk = B // BLK
    rows_a = NA * BLK
    ndis, idx_a = _dist_chunk(x, lm2, lmn, 0, NA, None)
    q_a = _make_sc_gather(rows_a)(lm, idx_a.reshape(rows_a))
    ndis, q_b = _dist_quant_chunk(x, lm2, lmn, lm, NA, nblk - NA, ndis)
    return jnp.concatenate([q_a, q_b], axis=0), ndis
